# baseline (speedup 1.0000x reference)
import functools
import math

import jax
import jax.numpy as jnp
from jax import lax
from jax.experimental import pallas as pl
from jax.experimental.pallas import tpu as pltpu
from jax.experimental.pallas import tpu_sc as plsc

D_MODEL = 64
SCALE = math.sqrt(D_MODEL)

NUM_CORES = 2
NUM_SUBCORES = 16
NUM_WORKERS = NUM_CORES * NUM_SUBCORES

CHUNK = 1024
GATHERS = CHUNK // 128


def _sc_embed(x2, table, pe2, n_rows, batch):
    n_per_w = n_rows // NUM_WORKERS
    n_chunks = n_per_w // CHUNK

    mesh = plsc.VectorSubcoreMesh(
        core_axis_name="c", subcore_axis_name="s",
        num_cores=NUM_CORES, num_subcores=NUM_SUBCORES,
    )

    @functools.partial(
        pl.kernel,
        mesh=mesh,
        compiler_params=pltpu.CompilerParams(use_tc_tiling_on_sc=False),
        out_type=jax.ShapeDtypeStruct((n_rows, D_MODEL), jnp.float32),
        scratch_types=[
            pltpu.VMEM((GATHERS, 128), jnp.int32),
            pltpu.VMEM((CHUNK, D_MODEL), jnp.float32),
            pltpu.VMEM((8, D_MODEL), jnp.float32),
            pltpu.SemaphoreType.DMA,
        ],
    )
    def sc_kernel(x_hbm, tbl_hbm, pe_hbm, out_hbm, idx_v, rows_v, pe_v, sem):
        wid = lax.axis_index("s") * NUM_CORES + lax.axis_index("c")
        base = wid * n_per_w

        def chunk_body(c, carry):
            row0 = pl.multiple_of(base + c * CHUNK, CHUNK)
            s_pos = row0 // batch
            pltpu.sync_copy(
                x_hbm.at[pl.ds(pl.multiple_of(row0 // 128, 8), GATHERS)], idx_v
            )
            pltpu.sync_copy(
                pe_hbm.at[pl.ds(pl.multiple_of((s_pos // 8) * 8, 8), 8)], pe_v
            )
            s_sub = s_pos % 8
            copies = [
                pltpu.async_copy(
                    tbl_hbm.at[idx_v.at[j]],
                    rows_v.at[pl.ds(j * 128, 128)],
                    sem,
                )
                for j in range(GATHERS)
            ]
            for cp in copies:
                cp.wait()
            pe_regs = [pe_v[s_sub, pl.ds(16 * j, 16)] for j in range(4)]

            def row_body(r, rcarry):
                for j in range(4):
                    v = rows_v[r, pl.ds(16 * j, 16)]
                    rows_v[r, pl.ds(16 * j, 16)] = v * SCALE + pe_regs[j]
                return rcarry

            lax.fori_loop(0, CHUNK, row_body, 0, unroll=4)
            pltpu.sync_copy(rows_v, out_hbm.at[pl.ds(row0, CHUNK)])
            return carry

        lax.fori_loop(0, n_chunks, chunk_body, 0)

    return sc_kernel(x2, table, pe2)


def kernel(x, table, pe):
    seq_len, batch = x.shape
    n_rows = seq_len * batch
    x2 = x.reshape(n_rows // 128, 128).astype(jnp.int32)
    pe2 = pe.reshape(-1, D_MODEL)
    out = _sc_embed(x2, table, pe2, n_rows, batch)
    return out.reshape(seq_len, batch, D_MODEL)
